# Initial kernel scaffold; baseline (speedup 1.0000x reference)
import jax
import jax.numpy as jnp
from jax.experimental import pallas as pl


def kernel(x_note, edge_index_a, edge_index_b, W_embed, b_embed, Wl1_a, Wr1_a, b1_a, Wl1_b, Wr1_b, b1_b, Wl2_a, Wr2_a, b2_a, Wl2_b, Wr2_b, b2_b, Wc1, bc1, Wc2, bc2):
    raise NotImplementedError("write your pallas kernel here")


# speedup vs baseline: 1.0121x; 1.0121x over previous
import jax
import jax.numpy as jnp
from jax.experimental import pallas as pl


def _embed_body(x_ref, w_ref, b_ref, o_ref):
    o_ref[...] = jnp.dot(x_ref[...], w_ref[...],
                         preferred_element_type=jnp.float32) + b_ref[...]


def _sage(x, edge_index, Wl, Wr, b):
    src = edge_index[0]
    dst = edge_index[1]
    msgs = jnp.take(x, src, axis=0)
    agg = jax.ops.segment_sum(msgs, dst, num_segments=x.shape[0])
    cnt = jax.ops.segment_sum(jnp.ones((edge_index.shape[1],), jnp.float32),
                              dst, num_segments=x.shape[0])
    mean = agg / jnp.maximum(cnt, 1.0)[:, None]
    return x @ Wl + mean @ Wr + b


def _coord_to_adj(edge_index, n):
    return jnp.zeros((n, n), jnp.float32).at[edge_index[0], edge_index[1]].set(1.0)


def _cluster(x, adjs, Wc, bc):
    A = adjs[0] + adjs[1]
    S = jax.nn.softmax(A @ (x @ Wc) + bc, axis=-1)
    x_p = S.T @ x
    adjs_p = (S.T @ adjs[0] @ S, S.T @ adjs[1] @ S)
    return x_p, adjs_p, S


def kernel(x_note, edge_index_a, edge_index_b, W_embed, b_embed, Wl1_a, Wr1_a, b1_a, Wl1_b, Wr1_b, b1_b, Wl2_a, Wr2_a, b2_a, Wl2_b, Wr2_b, b2_b, Wc1, bc1, Wc2, bc2):
    n = x_note.shape[0]
    h = pl.pallas_call(
        _embed_body,
        out_shape=jax.ShapeDtypeStruct((n, W_embed.shape[1]), jnp.float32),
    )(x_note, W_embed, b_embed)
    adj_a = _coord_to_adj(edge_index_a, n)
    adj_b = _coord_to_adj(edge_index_b, n)
    h1 = jax.nn.relu(_sage(h, edge_index_a, Wl1_a, Wr1_a, b1_a) +
                     _sage(h, edge_index_b, Wl1_b, Wr1_b, b1_b))
    h2 = jax.nn.relu(_sage(h1, edge_index_a, Wl2_a, Wr2_a, b2_a) +
                     _sage(h1, edge_index_b, Wl2_b, Wr2_b, b2_b))
    x1, adjs1, S1 = _cluster(h2, (adj_a, adj_b), Wc1, bc1)
    x2, adjs2, S2 = _cluster(x1, adjs1, Wc2, bc2)
    return (x2, S1, S2)


# speedup vs baseline: 3.6942x; 3.6501x over previous
import functools

import jax
import jax.numpy as jnp
from jax import lax
from jax.experimental import pallas as pl
from jax.experimental.pallas import tpu as pltpu
from jax.experimental.pallas import tpu_sc as plsc

N = 4096
E = 131072
D = 32
H = 256
NC = 2
NS = 16
LANES = 16
EPS = E // NS
G = 128
NG = EPS // G
HA = 128
DUMP = N
NACC = N + 256
ZW = 16
ZP = 128
KVAL = 15

_MESH = dict(core_axis_name="c", subcore_axis_name="s", num_cores=NC,
             num_subcores=NS)


def _sc1_body(src_hbm, dst_hbm, haug_hbm, zeros_hbm,
              agg_out, srcx_out, stamp_out,
              src1, dst1, got1, srcx1,
              keys2d, ids2d, dst2d, rows, acc_sh, sem):
    c = lax.axis_index("c")
    s = lax.axis_index("s")
    base = s * EPS
    off = c * E + base
    pltpu.sync_copy(src_hbm.at[pl.ds(off, EPS)], src1)
    pltpu.sync_copy(dst_hbm.at[pl.ds(off, EPS)], dst1)
    stamp_base = c * (N * N)

    def key_body(i, _):
        j = i // (G // LANES)
        co = (i % (G // LANES)) * LANES
        sl = pl.ds(i * LANES, LANES)
        sv = src1[sl]
        dv = dst1[sl]
        keys2d[j, pl.ds(co, LANES)] = stamp_base + sv * N + dv
        ids2d[j, pl.ds(co, LANES)] = base + i * LANES + lax.iota(jnp.int32,
                                                                 LANES)
        return 0
    lax.fori_loop(0, EPS // LANES, key_body, 0)

    def pack_body(j, _):
        pltpu.sync_copy(dst_hbm.at[pl.ds(off + j * G, G)], dst2d.at[j])
        return 0
    lax.fori_loop(0, NG, pack_body, 0)

    def scat_body(j, _):
        pltpu.sync_copy(ids2d.at[j], stamp_out.at[keys2d.at[j]])
        return 0
    lax.fori_loop(0, NG, scat_body, 0)

    rps = N // NS
    pltpu.sync_copy(zeros_hbm.at[pl.ds(s * rps, rps)],
                    acc_sh.at[pl.ds(s * rps, rps)])

    plsc.subcore_barrier()

    def gath_body(j, _):
        pltpu.sync_copy(stamp_out.at[keys2d.at[j]], got1.at[pl.ds(j * G, G)])
        return 0
    lax.fori_loop(0, NG, gath_body, 0)

    def srcx_body(i, _):
        j = i // (G // LANES)
        co = (i % (G // LANES)) * LANES
        sl = pl.ds(i * LANES, LANES)
        srcx1[sl] = jnp.where(got1[sl] == ids2d[j, pl.ds(co, LANES)],
                              src1[sl], DUMP)
        return 0
    lax.fori_loop(0, EPS // LANES, srcx_body, 0)
    pltpu.sync_copy(srcx1, srcx_out.at[pl.ds(off, EPS)])

    def agg_body(j, _):
        pltpu.sync_copy(haug_hbm.at[src1.at[pl.ds(j * G, G)]], rows)
        pltpu.async_copy(rows, acc_sh.at[dst2d.at[j]], sem, add=True).wait()
        return 0
    lax.fori_loop(0, NG, agg_body, 0)

    plsc.subcore_barrier()
    rows_per_sub = N // NS
    pltpu.sync_copy(acc_sh.at[pl.ds(s * rows_per_sub, rows_per_sub)],
                    agg_out.at[pl.ds(c * N + s * rows_per_sub, rows_per_sub)])


_sc1 = functools.partial(
    pl.kernel,
    out_type=(
        jax.ShapeDtypeStruct((NC * N, HA), jnp.float32),
        jax.ShapeDtypeStruct((NC * E,), jnp.int32),
        jax.ShapeDtypeStruct((NC * N * N,), jnp.int32),
    ),
    mesh=plsc.VectorSubcoreMesh(**_MESH),
    scratch_types=[
        pltpu.VMEM((EPS,), jnp.int32),
        pltpu.VMEM((EPS,), jnp.int32),
        pltpu.VMEM((EPS,), jnp.int32),
        pltpu.VMEM((EPS,), jnp.int32),
        pltpu.VMEM((NG, G), jnp.int32),
        pltpu.VMEM((NG, G), jnp.int32),
        pltpu.VMEM((NG, G), jnp.int32),
        pltpu.VMEM((G, HA), jnp.float32),
        pltpu.VMEM_SHARED((N, HA), jnp.float32),
        pltpu.SemaphoreType.DMA,
    ],
)(_sc1_body)


def _sc2_body(src_hbm, dst_hbm, h1lo_hbm, h1hi_hbm, zeros_hbm,
              agglo_out, agghi_out,
              src1, dst2d, rows, acc_sh, sem):
    c = lax.axis_index("c")
    s = lax.axis_index("s")
    base = s * EPS
    off = c * E + base
    pltpu.sync_copy(src_hbm.at[pl.ds(off, EPS)], src1)

    def pack_body(j, _):
        pltpu.sync_copy(dst_hbm.at[pl.ds(off + j * G, G)], dst2d.at[j])
        return 0
    lax.fori_loop(0, NG, pack_body, 0)

    rps = N // NS
    for h_hbm, out_hbm in ((h1lo_hbm, agglo_out), (h1hi_hbm, agghi_out)):
        pltpu.sync_copy(zeros_hbm.at[pl.ds(s * rps, rps)],
                        acc_sh.at[pl.ds(s * rps, rps)])
        plsc.subcore_barrier()

        def agg_body(j, _):
            pltpu.sync_copy(h_hbm.at[src1.at[pl.ds(j * G, G)]], rows)
            pltpu.async_copy(rows, acc_sh.at[dst2d.at[j]], sem,
                             add=True).wait()
            return 0
        lax.fori_loop(0, NG, agg_body, 0)

        plsc.subcore_barrier()
        pltpu.sync_copy(acc_sh.at[pl.ds(s * rps, rps)],
                        out_hbm.at[pl.ds(c * N + s * rps, rps)])
        plsc.subcore_barrier()


_sc2 = functools.partial(
    pl.kernel,
    out_type=(
        jax.ShapeDtypeStruct((NC * N, H // 2), jnp.float32),
        jax.ShapeDtypeStruct((NC * N, H // 2), jnp.float32),
    ),
    mesh=plsc.VectorSubcoreMesh(**_MESH),
    scratch_types=[
        pltpu.VMEM((EPS,), jnp.int32),
        pltpu.VMEM((NG, G), jnp.int32),
        pltpu.VMEM((G, H // 2), jnp.float32),
        pltpu.VMEM_SHARED((N, H // 2), jnp.float32),
        pltpu.SemaphoreType.DMA,
    ],
)(_sc2_body)


def _sc3_body(srcx_hbm, dst_hbm, z_hbm, zeros_hbm, out_hbm,
              dst1, sx2d, rows, acc_sh, sem):
    c = lax.axis_index("c")
    s = lax.axis_index("s")
    base = s * EPS
    off = c * E + base
    pltpu.sync_copy(dst_hbm.at[pl.ds(off, EPS)], dst1)

    def pack_body(j, _):
        pltpu.sync_copy(srcx_hbm.at[pl.ds(off + j * G, G)], sx2d.at[j])
        return 0
    lax.fori_loop(0, NG, pack_body, 0)

    acc_rows = NACC // NS
    pltpu.sync_copy(zeros_hbm.at[pl.ds(s * acc_rows, acc_rows)],
                    acc_sh.at[pl.ds(s * acc_rows, acc_rows)])

    plsc.subcore_barrier()

    def agg_body(j, _):
        pltpu.sync_copy(z_hbm.at[dst1.at[pl.ds(j * G, G)]], rows)
        pltpu.async_copy(rows, acc_sh.at[sx2d.at[j]], sem, add=True).wait()
        return 0
    lax.fori_loop(0, NG, agg_body, 0)

    plsc.subcore_barrier()
    rows_per_sub = N // NS
    pltpu.sync_copy(acc_sh.at[pl.ds(s * rows_per_sub, rows_per_sub)],
                    out_hbm.at[pl.ds(c * N + s * rows_per_sub, rows_per_sub)])


_sc3 = functools.partial(
    pl.kernel,
    out_type=jax.ShapeDtypeStruct((NC * N, ZP), jnp.float32),
    mesh=plsc.VectorSubcoreMesh(**_MESH),
    scratch_types=[
        pltpu.VMEM((EPS,), jnp.int32),
        pltpu.VMEM((NG, G), jnp.int32),
        pltpu.VMEM((G, ZP), jnp.float32),
        pltpu.VMEM_SHARED((NACC, ZP), jnp.float32),
        pltpu.SemaphoreType.DMA,
    ],
)(_sc3_body)


_BLK = 512
_GRID = N // _BLK


def _embed_body(x_ref, w_ref, b_ref, o_ref):
    h = jnp.dot(x_ref[...], w_ref[...],
                preferred_element_type=jnp.float32) + b_ref[...]
    ones = jnp.ones((h.shape[0], 1), jnp.float32)
    zer = jnp.zeros((h.shape[0], HA - D - 1), jnp.float32)
    o_ref[...] = jnp.concatenate([h, ones, zer], axis=1)


def _tc_embed(x, w, b):
    return pl.pallas_call(
        _embed_body,
        grid=(_GRID,),
        in_specs=[
            pl.BlockSpec((_BLK, x.shape[1]), lambda i: (i, 0)),
            pl.BlockSpec(w.shape, lambda i: (0, 0)),
            pl.BlockSpec(b.shape, lambda i: (0, 0)),
        ],
        out_specs=pl.BlockSpec((_BLK, HA), lambda i: (i, 0)),
        out_shape=jax.ShapeDtypeStruct((N, HA), jnp.float32),
    )(x, w, b)


def _sage_body(x_ref, agg_ref, wl_ref, wra_ref, wrb_ref, b_ref, olo_ref, ohi_ref, xw):
    x = x_ref[...][:, :xw]
    ga = agg_ref[0]
    gb = agg_ref[1]
    mean_a = ga[:, :xw] / jnp.maximum(ga[:, xw:xw + 1], 1.0)
    mean_b = gb[:, :xw] / jnp.maximum(gb[:, xw:xw + 1], 1.0)
    h1 = jax.nn.relu(
        jnp.dot(x, wl_ref[...], preferred_element_type=jnp.float32)
        + jnp.dot(mean_a, wra_ref[...], preferred_element_type=jnp.float32)
        + jnp.dot(mean_b, wrb_ref[...], preferred_element_type=jnp.float32)
        + b_ref[...])
    olo_ref[...] = h1[:, :H // 2]
    ohi_ref[...] = h1[:, H // 2:]


def _tc_sage1(haug, agg1, wl, wra, wrb, b):
    return pl.pallas_call(
        functools.partial(_sage_body, xw=D),
        grid=(_GRID,),
        in_specs=[
            pl.BlockSpec((_BLK, HA), lambda i: (i, 0)),
            pl.BlockSpec((NC, _BLK, HA), lambda i: (0, i, 0)),
            pl.BlockSpec(wl.shape, lambda i: (0, 0)),
            pl.BlockSpec(wra.shape, lambda i: (0, 0)),
            pl.BlockSpec(wrb.shape, lambda i: (0, 0)),
            pl.BlockSpec(b.shape, lambda i: (0, 0)),
        ],
        out_specs=[
            pl.BlockSpec((_BLK, H // 2), lambda i: (i, 0)),
            pl.BlockSpec((_BLK, H // 2), lambda i: (i, 0)),
        ],
        out_shape=[
            jax.ShapeDtypeStruct((N, H // 2), jnp.float32),
            jax.ShapeDtypeStruct((N, H // 2), jnp.float32),
        ],
    )(haug, agg1, wl, wra, wrb, b)


def _sage2_body(h1lo_ref, h1hi_ref, agglo_ref, agghi_ref, cnt_ref, wl_ref,
                wra_ref, wrb_ref, b_ref, wc_ref, h2_ref, z_ref):
    h1 = jnp.concatenate([h1lo_ref[...], h1hi_ref[...]], axis=1)
    agg_a = jnp.concatenate([agglo_ref[0], agghi_ref[0]], axis=1)
    agg_b = jnp.concatenate([agglo_ref[1], agghi_ref[1]], axis=1)
    mean_a = agg_a / jnp.maximum(cnt_ref[0][:, D:D + 1], 1.0)
    mean_b = agg_b / jnp.maximum(cnt_ref[1][:, D:D + 1], 1.0)
    h2 = jax.nn.relu(
        jnp.dot(h1, wl_ref[...], preferred_element_type=jnp.float32)
        + jnp.dot(mean_a, wra_ref[...], preferred_element_type=jnp.float32)
        + jnp.dot(mean_b, wrb_ref[...], preferred_element_type=jnp.float32)
        + b_ref[...])
    h2_ref[...] = h2
    z_ref[...] = jnp.dot(h2, wc_ref[...], preferred_element_type=jnp.float32)


def _tc_sage2(h1lo, h1hi, agg2lo, agg2hi, agg1, wl, wra, wrb, b, wc):
    return pl.pallas_call(
        _sage2_body,
        grid=(_GRID,),
        in_specs=[
            pl.BlockSpec((_BLK, H // 2), lambda i: (i, 0)),
            pl.BlockSpec((_BLK, H // 2), lambda i: (i, 0)),
            pl.BlockSpec((NC, _BLK, H // 2), lambda i: (0, i, 0)),
            pl.BlockSpec((NC, _BLK, H // 2), lambda i: (0, i, 0)),
            pl.BlockSpec((NC, _BLK, HA), lambda i: (0, i, 0)),
            pl.BlockSpec(wl.shape, lambda i: (0, 0)),
            pl.BlockSpec(wra.shape, lambda i: (0, 0)),
            pl.BlockSpec(wrb.shape, lambda i: (0, 0)),
            pl.BlockSpec(b.shape, lambda i: (0, 0)),
            pl.BlockSpec(wc.shape, lambda i: (0, 0)),
        ],
        out_specs=[
            pl.BlockSpec((_BLK, H), lambda i: (i, 0)),
            pl.BlockSpec((_BLK, ZP), lambda i: (i, 0)),
        ],
        out_shape=[
            jax.ShapeDtypeStruct((N, H), jnp.float32),
            jax.ShapeDtypeStruct((N, ZP), jnp.float32),
        ],
    )(h1lo, h1hi, agg2lo, agg2hi, agg1, wl, wra, wrb, b, wc)


def _softmax_body(za_ref, bc_ref, o_ref):
    logits = za_ref[0] + za_ref[1] + bc_ref[...]
    mask = lax.broadcasted_iota(jnp.int32, (1, ZP), 1) < KVAL
    logits = jnp.where(mask, logits, -1e30)
    m = jnp.max(logits, axis=-1, keepdims=True)
    e = jnp.exp(logits - m)
    o_ref[...] = e / jnp.sum(e, axis=-1, keepdims=True)


def _tc_softmax(za, bc):
    return pl.pallas_call(
        _softmax_body,
        grid=(_GRID,),
        in_specs=[
            pl.BlockSpec((NC, _BLK, ZP), lambda i: (0, i, 0)),
            pl.BlockSpec(bc.shape, lambda i: (0, 0)),
        ],
        out_specs=pl.BlockSpec((_BLK, ZP), lambda i: (i, 0)),
        out_shape=jax.ShapeDtypeStruct((N, ZP), jnp.float32),
    )(za, bc)


def _fin_body(s1_ref, h2_ref, t_ref, wc2_ref, bc2_ref, x2_ref, s2_ref,
              x1_acc, ga_acc, gb_acc):
    i = pl.program_id(0)

    @pl.when(i == 0)
    def _():
        x1_acc[...] = jnp.zeros((ZW, H), jnp.float32)
        ga_acc[...] = jnp.zeros((ZW, ZW), jnp.float32)
        gb_acc[...] = jnp.zeros((ZW, ZW), jnp.float32)

    s1 = s1_ref[...][:, :ZW]
    dn = (((0,), (0,)), ((), ()))
    x1_acc[...] += lax.dot_general(s1, h2_ref[...], dn,
                                   preferred_element_type=jnp.float32)
    ga_acc[...] += lax.dot_general(s1, t_ref[0][:, :ZW], dn,
                                   preferred_element_type=jnp.float32)
    gb_acc[...] += lax.dot_general(s1, t_ref[1][:, :ZW], dn,
                                   preferred_element_type=jnp.float32)

    @pl.when(i == pl.num_programs(0) - 1)
    def _():
        x1 = x1_acc[...]
        a1 = ga_acc[...] + gb_acc[...]
        l2 = jnp.dot(a1, jnp.dot(x1, wc2_ref[...],
                                 preferred_element_type=jnp.float32),
                     preferred_element_type=jnp.float32) + bc2_ref[...]
        mask = lax.broadcasted_iota(jnp.int32, (1, ZW), 1) < KVAL
        l2 = jnp.where(mask, l2, -1e30)
        m = jnp.max(l2, axis=-1, keepdims=True)
        e = jnp.exp(l2 - m)
        s2 = e / jnp.sum(e, axis=-1, keepdims=True)
        s2_ref[...] = s2
        x2_ref[...] = lax.dot_general(s2, x1, (((0,), (0,)), ((), ())),
                                      preferred_element_type=jnp.float32)


def _tc_final(s1p, h2, t, wc2, bc2):
    return pl.pallas_call(
        _fin_body,
        grid=(_GRID,),
        in_specs=[
            pl.BlockSpec((_BLK, ZP), lambda i: (i, 0)),
            pl.BlockSpec((_BLK, H), lambda i: (i, 0)),
            pl.BlockSpec((NC, _BLK, ZP), lambda i: (0, i, 0)),
            pl.BlockSpec(wc2.shape, lambda i: (0, 0)),
            pl.BlockSpec(bc2.shape, lambda i: (0, 0)),
        ],
        out_specs=[
            pl.BlockSpec((ZW, H), lambda i: (0, 0)),
            pl.BlockSpec((ZW, ZW), lambda i: (0, 0)),
        ],
        out_shape=[
            jax.ShapeDtypeStruct((ZW, H), jnp.float32),
            jax.ShapeDtypeStruct((ZW, ZW), jnp.float32),
        ],
        scratch_shapes=[
            pltpu.VMEM((ZW, H), jnp.float32),
            pltpu.VMEM((ZW, ZW), jnp.float32),
            pltpu.VMEM((ZW, ZW), jnp.float32),
        ],
    )(s1p, h2, t, wc2, bc2)


def kernel(x_note, edge_index_a, edge_index_b, W_embed, b_embed,
           Wl1_a, Wr1_a, b1_a, Wl1_b, Wr1_b, b1_b,
           Wl2_a, Wr2_a, b2_a, Wl2_b, Wr2_b, b2_b,
           Wc1, bc1, Wc2, bc2):
    K = Wc1.shape[1]
    src_ab = jnp.concatenate([edge_index_a[0], edge_index_b[0]])
    dst_ab = jnp.concatenate([edge_index_a[1], edge_index_b[1]])

    wl1 = Wl1_a + Wl1_b
    b1 = (b1_a + b1_b).reshape(1, H)
    wl2 = Wl2_a + Wl2_b
    b2 = (b2_a + b2_b).reshape(1, H)
    wc1p = jnp.pad(Wc1, ((0, 0), (0, ZP - K)))
    bc1p = jnp.pad(bc1, (0, ZP - K)).reshape(1, ZP)
    wc2p = jnp.pad(Wc2, ((0, 0), (0, ZW - K)))
    bc2p = jnp.pad(bc2, (0, ZW - K)).reshape(1, ZW)

    zeros1 = jnp.zeros((N, HA), jnp.float32)
    zeros3 = jnp.zeros((NACC, ZP), jnp.float32)

    haug = _tc_embed(x_note, W_embed, b_embed.reshape(1, D))
    agg1, srcx, _ = _sc1(src_ab, dst_ab, haug, zeros1)
    agg1r = agg1.reshape(NC, N, HA)
    h1lo, h1hi = _tc_sage1(haug, agg1r, wl1, Wr1_a, Wr1_b, b1)
    agg2lo, agg2hi = _sc2(src_ab, dst_ab, h1lo, h1hi, zeros1)
    h2, zp = _tc_sage2(h1lo, h1hi, agg2lo.reshape(NC, N, H // 2),
                       agg2hi.reshape(NC, N, H // 2), agg1r,
                       wl2, Wr2_a, Wr2_b, b2, wc1p)
    za = _sc3(srcx, dst_ab, zp, zeros3)
    s1p = _tc_softmax(za.reshape(NC, N, ZP), bc1p)
    t = _sc3(srcx, dst_ab, s1p, zeros3)
    x2p, s2p = _tc_final(s1p, h2, t.reshape(NC, N, ZP), wc2p, bc2p)
    return (x2p[:K], s1p[:, :K], s2p[:K, :K])


# speedup vs baseline: 5.1448x; 1.3927x over previous
import functools

import jax
import jax.numpy as jnp
from jax import lax
from jax.experimental import pallas as pl
from jax.experimental.pallas import tpu as pltpu
from jax.experimental.pallas import tpu_sc as plsc

N = 4096
E = 131072
D = 32
H = 256
NC = 2
NS = 16
LANES = 16
EPS = E // NS
G = 128
NG = EPS // G
HA = 128
ZPAD = N + 8
ZW = 16
ZP = 128
KVAL = 15

_MESH = dict(core_axis_name="c", subcore_axis_name="s", num_cores=NC,
             num_subcores=NS)


def _sc1_body(src_hbm, dst_hbm, haug_hbm, zeros_hbm,
              agg_out, srcx_out, stamp_out,
              src1, got1, srcx1,
              keys2d, ids2d, dst2d,
              rows0, rows1, acc_sh,
              semp, sems, semz, semg,
              sg0, sg1, sa0, sa1):
    c = lax.axis_index("c")
    s = lax.axis_index("s")
    base = s * EPS
    off = c * E + base
    pltpu.sync_copy(src_hbm.at[pl.ds(off, EPS)], src1)
    stamp_base = c * (N * N)

    rps = N // NS
    pltpu.async_copy(zeros_hbm.at[pl.ds(s * rps, rps)],
                     acc_sh.at[pl.ds(s * rps, rps)], semz)

    def pack_body(j, _):
        pltpu.sync_copy(dst_hbm.at[pl.ds(off + j * G, G)], dst2d.at[j])
        return 0
    lax.fori_loop(0, NG, pack_body, 0)

    def key_body(i, _):
        j = i // (G // LANES)
        co = (i % (G // LANES)) * LANES
        sl = pl.ds(i * LANES, LANES)
        sv = src1[sl]
        dv = dst2d[j, pl.ds(co, LANES)]
        keys2d[j, pl.ds(co, LANES)] = stamp_base + sv * N + dv
        ids2d[j, pl.ds(co, LANES)] = base + i * LANES + lax.iota(jnp.int32,
                                                                 LANES)
        return 0
    lax.fori_loop(0, EPS // LANES, key_body, 0)

    def scat_body(j, _):
        pltpu.sync_copy(ids2d.at[j], stamp_out.at[keys2d.at[j]])
        return 0
    lax.fori_loop(0, NG, scat_body, 0)
    pltpu.make_async_copy(zeros_hbm.at[pl.ds(s * rps, rps)],
                          acc_sh.at[pl.ds(s * rps, rps)], semz).wait()

    plsc.subcore_barrier()

    def gath_body(j, _):
        pltpu.sync_copy(stamp_out.at[keys2d.at[j]],
                        got1.at[pl.ds(j * G, G)])
        return 0
    lax.fori_loop(0, NG, gath_body, 0)

    rbufs = (rows0, rows1)
    sgs = (sg0, sg1)
    sas = (sa0, sa1)

    def gstart(j, t):
        pltpu.async_copy(haug_hbm.at[src1.at[pl.ds(j * G, G)]],
                         rbufs[t], sgs[t])

    def step(j, t):
        pltpu.make_async_copy(haug_hbm.at[src1.at[pl.ds(j * G, G)]],
                              rbufs[t], sgs[t]).wait()
        pltpu.async_copy(rbufs[t], acc_sh.at[dst2d.at[j]], sas[t],
                         add=True).wait()

    for t in range(2):
        gstart(t, t)

    def pipe_body(jj, _):
        for t in range(2):
            j = 2 * jj + t
            step(j, t)
            gstart(j + 2, t)
        return 0
    lax.fori_loop(0, NG // 2 - 1, pipe_body, 0)
    for t in range(2):
        step(NG - 2 + t, t)

    def srcx_body(i, _):
        j = i // (G // LANES)
        co = (i % (G // LANES)) * LANES
        sl = pl.ds(i * LANES, LANES)
        srcx1[sl] = jnp.where(got1[sl] == ids2d[j, pl.ds(co, LANES)],
                              src1[sl], -1)
        return 0
    lax.fori_loop(0, EPS // LANES, srcx_body, 0)
    pltpu.sync_copy(srcx1, srcx_out.at[pl.ds(off, EPS)])

    plsc.subcore_barrier()
    pltpu.sync_copy(acc_sh.at[pl.ds(s * rps, rps)],
                    agg_out.at[pl.ds(c * N + s * rps, rps)])


_sc1 = functools.partial(
    pl.kernel,
    out_type=(
        jax.ShapeDtypeStruct((NC * N, HA), jnp.float32),
        jax.ShapeDtypeStruct((NC * E,), jnp.int32),
        jax.ShapeDtypeStruct((NC * N * N,), jnp.int32),
    ),
    mesh=plsc.VectorSubcoreMesh(**_MESH),
    scratch_types=[
        pltpu.VMEM((EPS,), jnp.int32),
        pltpu.VMEM((EPS,), jnp.int32),
        pltpu.VMEM((EPS,), jnp.int32),
        pltpu.VMEM((NG, G), jnp.int32),
        pltpu.VMEM((NG, G), jnp.int32),
        pltpu.VMEM((NG, G), jnp.int32),
        pltpu.VMEM((G, HA), jnp.float32),
        pltpu.VMEM((G, HA), jnp.float32),
        pltpu.VMEM_SHARED((N, HA), jnp.float32),
    ] + [pltpu.SemaphoreType.DMA] * 8,
)(_sc1_body)


def _sc2_body(src_hbm, dst_hbm, h1lo_hbm, h1hi_hbm, zeros_hbm,
              agglo_out, agghi_out,
              src1, dst2d, rows0, rows1, rows2, rows3, acc_sh,
              semp, semz,
              sg0, sg1, sg2, sg3, sa0, sa1, sa2, sa3):
    c = lax.axis_index("c")
    s = lax.axis_index("s")
    base = s * EPS
    off = c * E + base
    pltpu.sync_copy(src_hbm.at[pl.ds(off, EPS)], src1)

    rps = N // NS
    pltpu.async_copy(zeros_hbm.at[pl.ds(s * rps, rps)],
                     acc_sh.at[pl.ds(s * rps, rps)], semz)

    def pack_body(j, _):
        pltpu.sync_copy(dst_hbm.at[pl.ds(off + j * G, G)], dst2d.at[j])
        return 0
    lax.fori_loop(0, NG, pack_body, 0)
    pltpu.make_async_copy(zeros_hbm.at[pl.ds(s * rps, rps)],
                          acc_sh.at[pl.ds(s * rps, rps)], semz).wait()

    plsc.subcore_barrier()

    rbufs = (rows0, rows1, rows2, rows3)
    sgs = (sg0, sg1, sg2, sg3)
    sas = (sa0, sa1, sa2, sa3)

    for half, (h_hbm, out_hbm) in enumerate(((h1lo_hbm, agglo_out),
                                             (h1hi_hbm, agghi_out))):
        def gstart(j, t):
            pltpu.async_copy(h_hbm.at[src1.at[pl.ds(j * G, G)]],
                             rbufs[t], sgs[t])

        def step(j, t):
            pltpu.make_async_copy(h_hbm.at[src1.at[pl.ds(j * G, G)]],
                                  rbufs[t], sgs[t]).wait()
            pltpu.async_copy(rbufs[t], acc_sh.at[dst2d.at[j]], sas[t],
                             add=True).wait()

        for t in range(4):
            gstart(t, t)

        def pipe_body(jj, _):
            for t in range(4):
                j = 4 * jj + t
                step(j, t)
                gstart(j + 4, t)
            return 0
        lax.fori_loop(0, NG // 4 - 1, pipe_body, 0)
        for t in range(4):
            step(NG - 4 + t, t)

        plsc.subcore_barrier()
        pltpu.sync_copy(acc_sh.at[pl.ds(s * rps, rps)],
                        out_hbm.at[pl.ds(c * N + s * rps, rps)])
        plsc.subcore_barrier()
        if half == 0:
            pltpu.sync_copy(zeros_hbm.at[pl.ds(s * rps, rps)],
                            acc_sh.at[pl.ds(s * rps, rps)])
            plsc.subcore_barrier()


_sc2 = functools.partial(
    pl.kernel,
    out_type=(
        jax.ShapeDtypeStruct((NC * N, H // 2), jnp.float32),
        jax.ShapeDtypeStruct((NC * N, H // 2), jnp.float32),
    ),
    mesh=plsc.VectorSubcoreMesh(**_MESH),
    scratch_types=[
        pltpu.VMEM((EPS,), jnp.int32),
        pltpu.VMEM((NG, G), jnp.int32),
        pltpu.VMEM((G, H // 2), jnp.float32),
        pltpu.VMEM((G, H // 2), jnp.float32),
        pltpu.VMEM((G, H // 2), jnp.float32),
        pltpu.VMEM((G, H // 2), jnp.float32),
        pltpu.VMEM_SHARED((N, H // 2), jnp.float32),
    ] + [pltpu.SemaphoreType.DMA] * 10,
)(_sc2_body)


def _sc3_body(srcx_hbm, dst_hbm, z_hbm, zeros_hbm, out_hbm,
              srcx1, dst1, gidx1, sx2d, rows0, rows1, rows2, rows3, acc_sh,
              semp, semz,
              sg0, sg1, sg2, sg3, sa0, sa1, sa2, sa3):
    c = lax.axis_index("c")
    s = lax.axis_index("s")
    base = s * EPS
    off = c * E + base
    pltpu.sync_copy(dst_hbm.at[pl.ds(off, EPS)], dst1)
    pltpu.sync_copy(srcx_hbm.at[pl.ds(off, EPS)], srcx1)

    rps = N // NS
    pltpu.async_copy(zeros_hbm.at[pl.ds(s * rps, rps)],
                     acc_sh.at[pl.ds(s * rps, rps)], semz)

    def idx_body(i, _):
        j = i // (G // LANES)
        co = (i % (G // LANES)) * LANES
        sl = pl.ds(i * LANES, LANES)
        sx = srcx1[sl]
        canon = sx >= 0
        gidx1[sl] = jnp.where(canon, dst1[sl], N)
        sx2d[j, pl.ds(co, LANES)] = jnp.where(canon, sx, 0)
        return 0
    lax.fori_loop(0, EPS // LANES, idx_body, 0)

    pltpu.make_async_copy(zeros_hbm.at[pl.ds(s * rps, rps)],
                          acc_sh.at[pl.ds(s * rps, rps)], semz).wait()
    plsc.subcore_barrier()

    rbufs = (rows0, rows1, rows2, rows3)
    sgs = (sg0, sg1, sg2, sg3)
    sas = (sa0, sa1, sa2, sa3)

    def gstart(j, t):
        pltpu.async_copy(z_hbm.at[gidx1.at[pl.ds(j * G, G)]],
                         rbufs[t], sgs[t])

    def step(j, t):
        pltpu.make_async_copy(z_hbm.at[gidx1.at[pl.ds(j * G, G)]],
                              rbufs[t], sgs[t]).wait()
        pltpu.async_copy(rbufs[t], acc_sh.at[sx2d.at[j]], sas[t],
                         add=True).wait()

    for t in range(4):
        gstart(t, t)

    def pipe_body(jj, _):
        for t in range(4):
            j = 4 * jj + t
            step(j, t)
            gstart(j + 4, t)
        return 0
    lax.fori_loop(0, NG // 4 - 1, pipe_body, 0)
    for t in range(4):
        step(NG - 4 + t, t)

    plsc.subcore_barrier()
    pltpu.sync_copy(acc_sh.at[pl.ds(s * rps, rps)],
                    out_hbm.at[pl.ds(c * N + s * rps, rps)])


_sc3 = functools.partial(
    pl.kernel,
    out_type=jax.ShapeDtypeStruct((NC * N, ZP), jnp.float32),
    mesh=plsc.VectorSubcoreMesh(**_MESH),
    scratch_types=[
        pltpu.VMEM((EPS,), jnp.int32),
        pltpu.VMEM((EPS,), jnp.int32),
        pltpu.VMEM((EPS,), jnp.int32),
        pltpu.VMEM((NG, G), jnp.int32),
        pltpu.VMEM((G, ZP), jnp.float32),
        pltpu.VMEM((G, ZP), jnp.float32),
        pltpu.VMEM((G, ZP), jnp.float32),
        pltpu.VMEM((G, ZP), jnp.float32),
        pltpu.VMEM_SHARED((N, ZP), jnp.float32),
    ] + [pltpu.SemaphoreType.DMA] * 10,
)(_sc3_body)


_BLK = 512
_GRID = N // _BLK


def _embed_body(x_ref, w_ref, b_ref, o_ref):
    h = jnp.dot(x_ref[...], w_ref[...],
                preferred_element_type=jnp.float32) + b_ref[...]
    ones = jnp.ones((h.shape[0], 1), jnp.float32)
    zer = jnp.zeros((h.shape[0], HA - D - 1), jnp.float32)
    o_ref[...] = jnp.concatenate([h, ones, zer], axis=1)


def _tc_embed(x, w, b):
    return pl.pallas_call(
        _embed_body,
        grid=(_GRID,),
        in_specs=[
            pl.BlockSpec((_BLK, x.shape[1]), lambda i: (i, 0)),
            pl.BlockSpec(w.shape, lambda i: (0, 0)),
            pl.BlockSpec(b.shape, lambda i: (0, 0)),
        ],
        out_specs=pl.BlockSpec((_BLK, HA), lambda i: (i, 0)),
        out_shape=jax.ShapeDtypeStruct((N, HA), jnp.float32),
    )(x, w, b)


def _sage_body(x_ref, agg_ref, wl_ref, wra_ref, wrb_ref, b_ref, olo_ref, ohi_ref, xw):
    x = x_ref[...][:, :xw]
    ga = agg_ref[0]
    gb = agg_ref[1]
    mean_a = ga[:, :xw] / jnp.maximum(ga[:, xw:xw + 1], 1.0)
    mean_b = gb[:, :xw] / jnp.maximum(gb[:, xw:xw + 1], 1.0)
    h1 = jax.nn.relu(
        jnp.dot(x, wl_ref[...], preferred_element_type=jnp.float32)
        + jnp.dot(mean_a, wra_ref[...], preferred_element_type=jnp.float32)
        + jnp.dot(mean_b, wrb_ref[...], preferred_element_type=jnp.float32)
        + b_ref[...])
    olo_ref[...] = h1[:, :H // 2]
    ohi_ref[...] = h1[:, H // 2:]


def _tc_sage1(haug, agg1, wl, wra, wrb, b):
    return pl.pallas_call(
        functools.partial(_sage_body, xw=D),
        grid=(_GRID,),
        in_specs=[
            pl.BlockSpec((_BLK, HA), lambda i: (i, 0)),
            pl.BlockSpec((NC, _BLK, HA), lambda i: (0, i, 0)),
            pl.BlockSpec(wl.shape, lambda i: (0, 0)),
            pl.BlockSpec(wra.shape, lambda i: (0, 0)),
            pl.BlockSpec(wrb.shape, lambda i: (0, 0)),
            pl.BlockSpec(b.shape, lambda i: (0, 0)),
        ],
        out_specs=[
            pl.BlockSpec((_BLK, H // 2), lambda i: (i, 0)),
            pl.BlockSpec((_BLK, H // 2), lambda i: (i, 0)),
        ],
        out_shape=[
            jax.ShapeDtypeStruct((N, H // 2), jnp.float32),
            jax.ShapeDtypeStruct((N, H // 2), jnp.float32),
        ],
    )(haug, agg1, wl, wra, wrb, b)


def _sage2_body(h1lo_ref, h1hi_ref, agglo_ref, agghi_ref, cnt_ref, wl_ref,
                wra_ref, wrb_ref, b_ref, wc_ref, h2_ref, z_ref):
    h1 = jnp.concatenate([h1lo_ref[...], h1hi_ref[...]], axis=1)
    agg_a = jnp.concatenate([agglo_ref[0], agghi_ref[0]], axis=1)
    agg_b = jnp.concatenate([agglo_ref[1], agghi_ref[1]], axis=1)
    mean_a = agg_a / jnp.maximum(cnt_ref[0][:, D:D + 1], 1.0)
    mean_b = agg_b / jnp.maximum(cnt_ref[1][:, D:D + 1], 1.0)
    h2 = jax.nn.relu(
        jnp.dot(h1, wl_ref[...], preferred_element_type=jnp.float32)
        + jnp.dot(mean_a, wra_ref[...], preferred_element_type=jnp.float32)
        + jnp.dot(mean_b, wrb_ref[...], preferred_element_type=jnp.float32)
        + b_ref[...])
    h2_ref[...] = h2
    z_ref[...] = jnp.dot(h2, wc_ref[...], preferred_element_type=jnp.float32)


def _tc_sage2(h1lo, h1hi, agg2lo, agg2hi, agg1, wl, wra, wrb, b, wc):
    return pl.pallas_call(
        _sage2_body,
        grid=(_GRID,),
        in_specs=[
            pl.BlockSpec((_BLK, H // 2), lambda i: (i, 0)),
            pl.BlockSpec((_BLK, H // 2), lambda i: (i, 0)),
            pl.BlockSpec((NC, _BLK, H // 2), lambda i: (0, i, 0)),
            pl.BlockSpec((NC, _BLK, H // 2), lambda i: (0, i, 0)),
            pl.BlockSpec((NC, _BLK, HA), lambda i: (0, i, 0)),
            pl.BlockSpec(wl.shape, lambda i: (0, 0)),
            pl.BlockSpec(wra.shape, lambda i: (0, 0)),
            pl.BlockSpec(wrb.shape, lambda i: (0, 0)),
            pl.BlockSpec(b.shape, lambda i: (0, 0)),
            pl.BlockSpec(wc.shape, lambda i: (0, 0)),
        ],
        out_specs=[
            pl.BlockSpec((_BLK, H), lambda i: (i, 0)),
            pl.BlockSpec((_BLK, ZP), lambda i: (i, 0)),
        ],
        out_shape=[
            jax.ShapeDtypeStruct((N, H), jnp.float32),
            jax.ShapeDtypeStruct((N, ZP), jnp.float32),
        ],
    )(h1lo, h1hi, agg2lo, agg2hi, agg1, wl, wra, wrb, b, wc)


def _softmax_body(za_ref, bc_ref, o_ref):
    logits = za_ref[0] + za_ref[1] + bc_ref[...]
    mask = lax.broadcasted_iota(jnp.int32, (1, ZP), 1) < KVAL
    logits = jnp.where(mask, logits, -1e30)
    m = jnp.max(logits, axis=-1, keepdims=True)
    e = jnp.exp(logits - m)
    o_ref[...] = e / jnp.sum(e, axis=-1, keepdims=True)


def _tc_softmax(za, bc):
    return pl.pallas_call(
        _softmax_body,
        grid=(_GRID,),
        in_specs=[
            pl.BlockSpec((NC, _BLK, ZP), lambda i: (0, i, 0)),
            pl.BlockSpec(bc.shape, lambda i: (0, 0)),
        ],
        out_specs=pl.BlockSpec((_BLK, ZP), lambda i: (i, 0)),
        out_shape=jax.ShapeDtypeStruct((N, ZP), jnp.float32),
    )(za, bc)


def _fin_body(s1_ref, h2_ref, t_ref, wc2_ref, bc2_ref, x2_ref, s2_ref,
              x1_acc, ga_acc, gb_acc):
    i = pl.program_id(0)

    @pl.when(i == 0)
    def _():
        x1_acc[...] = jnp.zeros((ZW, H), jnp.float32)
        ga_acc[...] = jnp.zeros((ZW, ZW), jnp.float32)
        gb_acc[...] = jnp.zeros((ZW, ZW), jnp.float32)

    s1 = s1_ref[...][:, :ZW]
    dn = (((0,), (0,)), ((), ()))
    x1_acc[...] += lax.dot_general(s1, h2_ref[...], dn,
                                   preferred_element_type=jnp.float32)
    ga_acc[...] += lax.dot_general(s1, t_ref[0][:, :ZW], dn,
                                   preferred_element_type=jnp.float32)
    gb_acc[...] += lax.dot_general(s1, t_ref[1][:, :ZW], dn,
                                   preferred_element_type=jnp.float32)

    @pl.when(i == pl.num_programs(0) - 1)
    def _():
        x1 = x1_acc[...]
        a1 = ga_acc[...] + gb_acc[...]
        l2 = jnp.dot(a1, jnp.dot(x1, wc2_ref[...],
                                 preferred_element_type=jnp.float32),
                     preferred_element_type=jnp.float32) + bc2_ref[...]
        mask = lax.broadcasted_iota(jnp.int32, (1, ZW), 1) < KVAL
        l2 = jnp.where(mask, l2, -1e30)
        m = jnp.max(l2, axis=-1, keepdims=True)
        e = jnp.exp(l2 - m)
        s2 = e / jnp.sum(e, axis=-1, keepdims=True)
        s2_ref[...] = s2
        x2_ref[...] = lax.dot_general(s2, x1, (((0,), (0,)), ((), ())),
                                      preferred_element_type=jnp.float32)


def _tc_final(s1p, h2, t, wc2, bc2):
    return pl.pallas_call(
        _fin_body,
        grid=(_GRID,),
        in_specs=[
            pl.BlockSpec((_BLK, ZP), lambda i: (i, 0)),
            pl.BlockSpec((_BLK, H), lambda i: (i, 0)),
            pl.BlockSpec((NC, _BLK, ZP), lambda i: (0, i, 0)),
            pl.BlockSpec(wc2.shape, lambda i: (0, 0)),
            pl.BlockSpec(bc2.shape, lambda i: (0, 0)),
        ],
        out_specs=[
            pl.BlockSpec((ZW, H), lambda i: (0, 0)),
            pl.BlockSpec((ZW, ZW), lambda i: (0, 0)),
        ],
        out_shape=[
            jax.ShapeDtypeStruct((ZW, H), jnp.float32),
            jax.ShapeDtypeStruct((ZW, ZW), jnp.float32),
        ],
        scratch_shapes=[
            pltpu.VMEM((ZW, H), jnp.float32),
            pltpu.VMEM((ZW, ZW), jnp.float32),
            pltpu.VMEM((ZW, ZW), jnp.float32),
        ],
    )(s1p, h2, t, wc2, bc2)


def kernel(x_note, edge_index_a, edge_index_b, W_embed, b_embed,
           Wl1_a, Wr1_a, b1_a, Wl1_b, Wr1_b, b1_b,
           Wl2_a, Wr2_a, b2_a, Wl2_b, Wr2_b, b2_b,
           Wc1, bc1, Wc2, bc2):
    K = Wc1.shape[1]
    src_ab = jnp.concatenate([edge_index_a[0], edge_index_b[0]])
    dst_ab = jnp.concatenate([edge_index_a[1], edge_index_b[1]])

    wl1 = Wl1_a + Wl1_b
    b1 = (b1_a + b1_b).reshape(1, H)
    wl2 = Wl2_a + Wl2_b
    b2 = (b2_a + b2_b).reshape(1, H)
    wc1p = jnp.pad(Wc1, ((0, 0), (0, ZP - K)))
    bc1p = jnp.pad(bc1, (0, ZP - K)).reshape(1, ZP)
    wc2p = jnp.pad(Wc2, ((0, 0), (0, ZW - K)))
    bc2p = jnp.pad(bc2, (0, ZW - K)).reshape(1, ZW)

    zeros1 = jnp.zeros((N, HA), jnp.float32)
    zeros3 = jnp.zeros((N, ZP), jnp.float32)

    haug = _tc_embed(x_note, W_embed, b_embed.reshape(1, D))
    agg1, srcx, _ = _sc1(src_ab, dst_ab, haug, zeros1)
    agg1r = agg1.reshape(NC, N, HA)
    h1lo, h1hi = _tc_sage1(haug, agg1r, wl1, Wr1_a, Wr1_b, b1)
    agg2lo, agg2hi = _sc2(src_ab, dst_ab, h1lo, h1hi, zeros1)
    h2, zp = _tc_sage2(h1lo, h1hi, agg2lo.reshape(NC, N, H // 2),
                       agg2hi.reshape(NC, N, H // 2), agg1r,
                       wl2, Wr2_a, Wr2_b, b2, wc1p)
    zpp = jnp.pad(zp, ((0, ZPAD - N), (0, 0)))
    za = _sc3(srcx, dst_ab, zpp, zeros3)
    s1p = _tc_softmax(za.reshape(NC, N, ZP), bc1p)
    s1pp = jnp.pad(s1p, ((0, ZPAD - N), (0, 0)))
    t = _sc3(srcx, dst_ab, s1pp, zeros3)
    x2p, s2p = _tc_final(s1p, h2, t.reshape(NC, N, ZP), wc2p, bc2p)
    return (x2p[:K], s1p[:, :K], s2p[:K, :K])
